# baseline (speedup 1.0000x reference)
import functools

import jax
import jax.numpy as jnp
from jax import lax
from jax.experimental import pallas as pl
from jax.experimental.pallas import tpu as pltpu
from jax.experimental.pallas import tpu_sc as plsc

E = 320000
H = 128
D0, D1, D2 = 5, 6, 2
NT = D0 * D1 * D2
NTP = 64
NC, NS, L = 2, 16, 16
NW = NC * NS
U = 128
NU = E // U


GROUP = 6
NJ_MAIN = 78
NU_TAIL = NU - NW * NJ_MAIN


def _sc_body(a0_hbm, a1_hbm, a2_hbm, w0_hbm, w1_hbm, w2_hbm, out_hbm, t_hbm,
             w0_v, w1_v, w2_v, t_v, a0_v, a1_v, a2_v, code_v, rows_v,
             gsem, ssem):
    c = lax.axis_index("c")
    s = lax.axis_index("s")
    wid = s * NC + c

    @pl.when(s == 0)
    def _build():
        pltpu.sync_copy(w0_hbm, w0_v)
        pltpu.sync_copy(w1_hbm, w1_v)
        pltpu.sync_copy(w2_hbm, w2_v)
        third = jnp.float32(1.0 / 3.0)
        for i0 in range(D0):
            for i1 in range(D1):
                for j in range(H // L):
                    sl = pl.ds(j * L, L)
                    s01 = w0_v[i0, sl] + w1_v[i1, sl]
                    r = (i0 * D1 + i1) * D2
                    t_v[r, sl] = (s01 + w2_v[0, sl]) * third
                    t_v[r + 1, sl] = (s01 + w2_v[1, sl]) * third
        for r in range(NT, NTP):
            for j in range(H // L):
                t_v[r, pl.ds(j * L, L)] = jnp.zeros((L,), jnp.float32)
        pltpu.sync_copy(t_v, t_hbm.at[pl.ds(c * NTP, NTP)])

    plsc.subcore_barrier()

    t_base = c * NTP
    u_base = wid * NJ_MAIN

    @pl.loop(0, NJ_MAIN, step=GROUP)
    def _group(g0):
        base_e = (u_base + g0) * U
        pltpu.sync_copy(a0_hbm.at[pl.ds(base_e, GROUP * U)], a0_v)
        pltpu.sync_copy(a1_hbm.at[pl.ds(base_e, GROUP * U)], a1_v)
        pltpu.sync_copy(a2_hbm.at[pl.ds(base_e, GROUP * U)], a2_v)
        for b in range(GROUP):
            for k in range(U // L):
                sl = pl.ds(b * U + k * L, L)
                code_v[b, pl.ds(k * L, L)] = (
                    a0_v[sl] * (D1 * D2) + a1_v[sl] * D2 + a2_v[sl]) + t_base
        gathers = []
        for b in range(GROUP):
            @pl.when(g0 > 0)
            def _drain():
                pltpu.make_async_copy(
                    rows_v.at[b], out_hbm.at[pl.ds(base_e, U)],
                    ssem.at[b]).wait()
            gathers.append(pltpu.async_copy(
                t_hbm.at[code_v.at[b]], rows_v.at[b], gsem.at[b]))
        for b in range(GROUP):
            gathers[b].wait()
            pltpu.async_copy(
                rows_v.at[b], out_hbm.at[pl.ds(base_e + b * U, U)],
                ssem.at[b])

    for b in range(GROUP):
        pltpu.make_async_copy(
            rows_v.at[b], out_hbm.at[pl.ds(0, U)], ssem.at[b]).wait()

    @pl.when(wid < NU_TAIL)
    def _tail():
        base_e = (NW * NJ_MAIN + wid) * U
        pltpu.sync_copy(a0_hbm.at[pl.ds(base_e, U)], a0_v.at[pl.ds(0, U)])
        pltpu.sync_copy(a1_hbm.at[pl.ds(base_e, U)], a1_v.at[pl.ds(0, U)])
        pltpu.sync_copy(a2_hbm.at[pl.ds(base_e, U)], a2_v.at[pl.ds(0, U)])
        for k in range(U // L):
            sl = pl.ds(k * L, L)
            code_v[0, sl] = (
                a0_v[sl] * (D1 * D2) + a1_v[sl] * D2 + a2_v[sl]) + t_base
        pltpu.async_copy(t_hbm.at[code_v.at[0]], rows_v.at[0], gsem.at[0]).wait()
        pltpu.sync_copy(rows_v.at[0], out_hbm.at[pl.ds(base_e, U)])


_launch = functools.partial(
    pl.kernel,
    out_type=(
        jax.ShapeDtypeStruct((E, H), jnp.float32),
        jax.ShapeDtypeStruct((NC * NTP, H), jnp.float32),
    ),
    mesh=plsc.VectorSubcoreMesh(core_axis_name="c", subcore_axis_name="s"),
    scratch_types=[
        pltpu.VMEM((D0, H), jnp.float32),
        pltpu.VMEM((D1, H), jnp.float32),
        pltpu.VMEM((D2, H), jnp.float32),
        pltpu.VMEM((NTP, H), jnp.float32),
        pltpu.VMEM((GROUP * U,), jnp.int32),
        pltpu.VMEM((GROUP * U,), jnp.int32),
        pltpu.VMEM((GROUP * U,), jnp.int32),
        pltpu.VMEM((GROUP, U), jnp.int32),
        pltpu.VMEM((GROUP, U, H), jnp.float32),
        pltpu.SemaphoreType.DMA((GROUP,)),
        pltpu.SemaphoreType.DMA((GROUP,)),
    ],
)(_sc_body)


@jax.jit
def kernel(edge_attr, W0, W1, W2):
    ea_t = edge_attr.T
    out, _ = _launch(ea_t[0], ea_t[1], ea_t[2], W0, W1, W2)
    return out


# speedup vs baseline: 12.4131x; 12.4131x over previous
import functools

import jax
import jax.numpy as jnp
from jax import lax
from jax.experimental import pallas as pl
from jax.experimental.pallas import tpu as pltpu
from jax.experimental.pallas import tpu_sc as plsc

E = 320000
H = 128
D0, D1, D2 = 5, 6, 2
NT = D0 * D1 * D2
NTP = 64
NC, NS, L = 2, 16, 16
NW = NC * NS
U = 128
NU = E // U
GROUP = 6
NJ_MAIN = 78
NU_TAIL = NU - NW * NJ_MAIN


def _sc_body(a0_hbm, a1_hbm, a2_hbm, w0_hbm, w1_hbm, w2_hbm, out_hbm,
             w0_v, w1_v, w2_v, t_v, t_sh, a0_v, a1_v, a2_v, code_v, rows_v,
             gsem, ssem):
    c = lax.axis_index("c")
    s = lax.axis_index("s")
    wid = s * NC + c

    @pl.when(s == 0)
    def _build():
        pltpu.sync_copy(w0_hbm, w0_v)
        pltpu.sync_copy(w1_hbm, w1_v)
        pltpu.sync_copy(w2_hbm, w2_v)
        third = jnp.float32(1.0 / 3.0)
        for i0 in range(D0):
            for i1 in range(D1):
                for j in range(H // L):
                    sl = pl.ds(j * L, L)
                    s01 = w0_v[i0, sl] + w1_v[i1, sl]
                    r = (i0 * D1 + i1) * D2
                    t_v[r, sl] = (s01 + w2_v[0, sl]) * third
                    t_v[r + 1, sl] = (s01 + w2_v[1, sl]) * third
        pltpu.sync_copy(t_v, t_sh)

    plsc.subcore_barrier()

    u_base = wid * NJ_MAIN

    def pack_codes(b, off):
        for k in range(U // L):
            sl = pl.ds(off + k * L, L)
            code_v[b, pl.ds(k * L, L)] = (
                a0_v[sl] * (D1 * D2) + a1_v[sl] * D2 + a2_v[sl])

    @pl.loop(0, NJ_MAIN, step=GROUP)
    def _group(g0):
        base_e = (u_base + g0) * U
        pltpu.sync_copy(a0_hbm.at[pl.ds(base_e, GROUP * U)], a0_v)
        pltpu.sync_copy(a1_hbm.at[pl.ds(base_e, GROUP * U)], a1_v)
        pltpu.sync_copy(a2_hbm.at[pl.ds(base_e, GROUP * U)], a2_v)
        for b in range(GROUP):
            pack_codes(b, b * U)
        gathers = []
        for b in range(GROUP):
            @pl.when(g0 > 0)
            def _drain():
                pltpu.make_async_copy(
                    rows_v.at[b], out_hbm.at[pl.ds(base_e, U)],
                    ssem.at[b]).wait()
            gathers.append(pltpu.async_copy(
                t_sh.at[code_v.at[b]], rows_v.at[b], gsem.at[b]))
        for b in range(GROUP):
            gathers[b].wait()
            pltpu.async_copy(
                rows_v.at[b], out_hbm.at[pl.ds(base_e + b * U, U)],
                ssem.at[b])

    for b in range(GROUP):
        pltpu.make_async_copy(
            rows_v.at[b], out_hbm.at[pl.ds(0, U)], ssem.at[b]).wait()

    @pl.when(wid < NU_TAIL)
    def _tail():
        base_e = (NW * NJ_MAIN + wid) * U
        pltpu.sync_copy(a0_hbm.at[pl.ds(base_e, U)], a0_v.at[pl.ds(0, U)])
        pltpu.sync_copy(a1_hbm.at[pl.ds(base_e, U)], a1_v.at[pl.ds(0, U)])
        pltpu.sync_copy(a2_hbm.at[pl.ds(base_e, U)], a2_v.at[pl.ds(0, U)])
        pack_codes(0, 0)
        pltpu.async_copy(t_sh.at[code_v.at[0]], rows_v.at[0], gsem.at[0]).wait()
        pltpu.sync_copy(rows_v.at[0], out_hbm.at[pl.ds(base_e, U)])


_launch = functools.partial(
    pl.kernel,
    out_type=jax.ShapeDtypeStruct((E, H), jnp.float32),
    mesh=plsc.VectorSubcoreMesh(core_axis_name="c", subcore_axis_name="s"),
    scratch_types=[
        pltpu.VMEM((D0, H), jnp.float32),
        pltpu.VMEM((D1, H), jnp.float32),
        pltpu.VMEM((D2, H), jnp.float32),
        pltpu.VMEM((NTP, H), jnp.float32),
        pltpu.VMEM_SHARED((NTP, H), jnp.float32),
        pltpu.VMEM((GROUP * U,), jnp.int32),
        pltpu.VMEM((GROUP * U,), jnp.int32),
        pltpu.VMEM((GROUP * U,), jnp.int32),
        pltpu.VMEM((GROUP, U), jnp.int32),
        pltpu.VMEM((GROUP, U, H), jnp.float32),
        pltpu.SemaphoreType.DMA((GROUP,)),
        pltpu.SemaphoreType.DMA((GROUP,)),
    ],
)(_sc_body)


@jax.jit
def kernel(edge_attr, W0, W1, W2):
    ea_t = edge_attr.T
    return _launch(ea_t[0], ea_t[1], ea_t[2], W0, W1, W2)


# speedup vs baseline: 15.3198x; 1.2342x over previous
import functools

import jax
import jax.numpy as jnp
from jax import lax
from jax.experimental import pallas as pl
from jax.experimental.pallas import tpu as pltpu
from jax.experimental.pallas import tpu_sc as plsc

E = 320000
H = 128
D0, D1, D2 = 5, 6, 2
NT = D0 * D1 * D2
NTP = 64
NC, NS, L = 2, 16, 16
NW = NC * NS
U = 128
NU = E // U
GROUP = 6
NJ_MAIN = 78
NU_TAIL = NU - NW * NJ_MAIN


def _sc_body(a0_hbm, a1_hbm, a2_hbm, w0_hbm, w1_hbm, w2_hbm, out_hbm,
             w0_v, w1_v, w2_v, t_v, t_sh, a0_v, a1_v, a2_v, code_v, rows_v,
             gsem, ssem, csem):
    c = lax.axis_index("c")
    s = lax.axis_index("s")
    wid = s * NC + c

    @pl.when(s == 0)
    def _build():
        pltpu.sync_copy(w0_hbm, w0_v)
        pltpu.sync_copy(w1_hbm, w1_v)
        pltpu.sync_copy(w2_hbm, w2_v)
        third = jnp.float32(1.0 / 3.0)
        for i0 in range(D0):
            for i1 in range(D1):
                for j in range(H // L):
                    sl = pl.ds(j * L, L)
                    s01 = w0_v[i0, sl] + w1_v[i1, sl]
                    r = (i0 * D1 + i1) * D2
                    t_v[r, sl] = (s01 + w2_v[0, sl]) * third
                    t_v[r + 1, sl] = (s01 + w2_v[1, sl]) * third
        pltpu.sync_copy(t_v, t_sh)

    plsc.subcore_barrier()

    u_base = wid * NJ_MAIN
    GU = GROUP * U

    def pack_codes(b, off):
        for k in range(U // L):
            sl = pl.ds(off + k * L, L)
            code_v[b, pl.ds(k * L, L)] = (
                a0_v[sl] * (D1 * D2) + a1_v[sl] * D2 + a2_v[sl])

    def start_cols(base_e, half, par):
        for col_hbm, col_v in ((a0_hbm, a0_v), (a1_hbm, a1_v), (a2_hbm, a2_v)):
            pltpu.async_copy(col_hbm.at[pl.ds(base_e, GU)],
                             col_v.at[pl.ds(half, GU)], csem.at[par])

    def wait_cols(par):
        for col_hbm, col_v in ((a0_hbm, a0_v), (a1_hbm, a1_v), (a2_hbm, a2_v)):
            pltpu.make_async_copy(col_hbm.at[pl.ds(0, GU)],
                                  col_v.at[pl.ds(0, GU)], csem.at[par]).wait()

    start_cols(u_base * U, 0, 0)

    @pl.loop(0, NJ_MAIN, step=GROUP)
    def _group(g0):
        base_e = (u_base + g0) * U
        par = lax.rem(lax.div(g0, jnp.int32(GROUP)), jnp.int32(2))
        half = pl.multiple_of(par * GU, 8)
        wait_cols(par)

        @pl.when(g0 + GROUP < NJ_MAIN)
        def _prefetch():
            nxt = pl.multiple_of((1 - par) * GU, 8)
            start_cols(base_e + GU, nxt, 1 - par)

        for b in range(GROUP):
            pack_codes(b, half + b * U)
        gathers = []
        for b in range(GROUP):
            @pl.when(g0 > 0)
            def _drain():
                pltpu.make_async_copy(
                    rows_v.at[b], out_hbm.at[pl.ds(base_e, U)],
                    ssem.at[b]).wait()
            gathers.append(pltpu.async_copy(
                t_sh.at[code_v.at[b]], rows_v.at[b], gsem.at[b]))
        for b in range(GROUP):
            gathers[b].wait()
            pltpu.async_copy(
                rows_v.at[b], out_hbm.at[pl.ds(base_e + b * U, U)],
                ssem.at[b])

    for b in range(GROUP):
        pltpu.make_async_copy(
            rows_v.at[b], out_hbm.at[pl.ds(0, U)], ssem.at[b]).wait()

    @pl.when(wid < NU_TAIL)
    def _tail():
        base_e = (NW * NJ_MAIN + wid) * U
        pltpu.sync_copy(a0_hbm.at[pl.ds(base_e, U)], a0_v.at[pl.ds(0, U)])
        pltpu.sync_copy(a1_hbm.at[pl.ds(base_e, U)], a1_v.at[pl.ds(0, U)])
        pltpu.sync_copy(a2_hbm.at[pl.ds(base_e, U)], a2_v.at[pl.ds(0, U)])
        pack_codes(0, 0)
        pltpu.async_copy(t_sh.at[code_v.at[0]], rows_v.at[0], gsem.at[0]).wait()
        pltpu.sync_copy(rows_v.at[0], out_hbm.at[pl.ds(base_e, U)])


_launch = functools.partial(
    pl.kernel,
    out_type=jax.ShapeDtypeStruct((E, H), jnp.float32),
    mesh=plsc.VectorSubcoreMesh(core_axis_name="c", subcore_axis_name="s"),
    scratch_types=[
        pltpu.VMEM((D0, H), jnp.float32),
        pltpu.VMEM((D1, H), jnp.float32),
        pltpu.VMEM((D2, H), jnp.float32),
        pltpu.VMEM((NTP, H), jnp.float32),
        pltpu.VMEM_SHARED((NTP, H), jnp.float32),
        pltpu.VMEM((2 * GROUP * U,), jnp.int32),
        pltpu.VMEM((2 * GROUP * U,), jnp.int32),
        pltpu.VMEM((2 * GROUP * U,), jnp.int32),
        pltpu.VMEM((GROUP, U), jnp.int32),
        pltpu.VMEM((GROUP, U, H), jnp.float32),
        pltpu.SemaphoreType.DMA((GROUP,)),
        pltpu.SemaphoreType.DMA((GROUP,)),
        pltpu.SemaphoreType.DMA((2,)),
    ],
)(_sc_body)


@jax.jit
def kernel(edge_attr, W0, W1, W2):
    ea_t = edge_attr.T
    return _launch(ea_t[0], ea_t[1], ea_t[2], W0, W1, W2)


# speedup vs baseline: 15.3622x; 1.0028x over previous
import functools

import jax
import jax.numpy as jnp
from jax import lax
from jax.experimental import pallas as pl
from jax.experimental.pallas import tpu as pltpu
from jax.experimental.pallas import tpu_sc as plsc

E = 320000
H = 128
D0, D1, D2 = 5, 6, 2
NT = D0 * D1 * D2
NTP = 64
NC, NS, L = 2, 16, 16
NW = NC * NS
U = 128
NU = E // U
GROUP = 6
NJ_MAIN = 78
NU_TAIL = NU - NW * NJ_MAIN


def _sc_body(a0_hbm, a1_hbm, a2_hbm, w0_hbm, w1_hbm, w2_hbm, out_hbm,
             w0_v, w1_v, w2_v, t_v, t_sh, a0_v, a1_v, a2_v, code_v, rows_v,
             gsem, ssem, csem):
    c = lax.axis_index("c")
    s = lax.axis_index("s")
    wid = s * NC + c

    @pl.when(s == 0)
    def _build():
        pltpu.sync_copy(w0_hbm, w0_v)
        pltpu.sync_copy(w1_hbm, w1_v)
        pltpu.sync_copy(w2_hbm, w2_v)
        third = jnp.float32(1.0 / 3.0)
        for i0 in range(D0):
            for i1 in range(D1):
                for j in range(H // L):
                    sl = pl.ds(j * L, L)
                    s01 = w0_v[i0, sl] + w1_v[i1, sl]
                    r = (i0 * D1 + i1) * D2
                    t_v[r, sl] = (s01 + w2_v[0, sl]) * third
                    t_v[r + 1, sl] = (s01 + w2_v[1, sl]) * third
        pltpu.sync_copy(t_v, t_sh)

    plsc.subcore_barrier()

    u_base = wid * NJ_MAIN
    GU = GROUP * U

    def pack_codes(b, off):
        for k in range(U // L):
            sl = pl.ds(off + k * L, L)
            code_v[b, pl.ds(k * L, L)] = (
                a0_v[sl] * (D1 * D2) + a1_v[sl] * D2 + a2_v[sl])

    def start_cols(base_e, half, par):
        for col_hbm, col_v in ((a0_hbm, a0_v), (a1_hbm, a1_v), (a2_hbm, a2_v)):
            pltpu.async_copy(col_hbm.at[pl.ds(base_e, GU)],
                             col_v.at[pl.ds(half, GU)], csem.at[par])

    def wait_cols(par):
        for col_hbm, col_v in ((a0_hbm, a0_v), (a1_hbm, a1_v), (a2_hbm, a2_v)):
            pltpu.make_async_copy(col_hbm.at[pl.ds(0, GU)],
                                  col_v.at[pl.ds(0, GU)], csem.at[par]).wait()

    start_cols(u_base * U, 0, 0)

    @pl.loop(0, NJ_MAIN, step=GROUP)
    def _group(g0):
        base_e = (u_base + g0) * U
        par = lax.rem(lax.div(g0, jnp.int32(GROUP)), jnp.int32(2))
        half = pl.multiple_of(par * GU, 8)
        wait_cols(par)

        @pl.when(g0 + GROUP < NJ_MAIN)
        def _prefetch():
            nxt = pl.multiple_of((1 - par) * GU, 8)
            start_cols(base_e + GU, nxt, 1 - par)

        for b in range(GROUP):
            pack_codes(b, half + b * U)
        gathers = []
        for p in range(GROUP // 2):
            @pl.when(g0 > 0)
            def _drain():
                pltpu.make_async_copy(
                    rows_v.at[p], out_hbm.at[pl.ds(base_e, 2 * U)],
                    ssem.at[p]).wait()
            for h in range(2):
                b = 2 * p + h
                gathers.append(pltpu.async_copy(
                    t_sh.at[code_v.at[b]],
                    rows_v.at[p].at[pl.ds(h * U, U)], gsem.at[b]))
        for p in range(GROUP // 2):
            gathers[2 * p].wait()
            gathers[2 * p + 1].wait()
            pltpu.async_copy(
                rows_v.at[p], out_hbm.at[pl.ds(base_e + 2 * p * U, 2 * U)],
                ssem.at[p])

    for p in range(GROUP // 2):
        pltpu.make_async_copy(
            rows_v.at[p], out_hbm.at[pl.ds(0, 2 * U)], ssem.at[p]).wait()

    @pl.when(wid < NU_TAIL)
    def _tail():
        base_e = (NW * NJ_MAIN + wid) * U
        pltpu.sync_copy(a0_hbm.at[pl.ds(base_e, U)], a0_v.at[pl.ds(0, U)])
        pltpu.sync_copy(a1_hbm.at[pl.ds(base_e, U)], a1_v.at[pl.ds(0, U)])
        pltpu.sync_copy(a2_hbm.at[pl.ds(base_e, U)], a2_v.at[pl.ds(0, U)])
        pack_codes(0, 0)
        pltpu.async_copy(t_sh.at[code_v.at[0]],
                         rows_v.at[0].at[pl.ds(0, U)], gsem.at[0]).wait()
        pltpu.sync_copy(rows_v.at[0].at[pl.ds(0, U)],
                        out_hbm.at[pl.ds(base_e, U)])


_launch = functools.partial(
    pl.kernel,
    out_type=jax.ShapeDtypeStruct((E, H), jnp.float32),
    mesh=plsc.VectorSubcoreMesh(core_axis_name="c", subcore_axis_name="s"),
    scratch_types=[
        pltpu.VMEM((D0, H), jnp.float32),
        pltpu.VMEM((D1, H), jnp.float32),
        pltpu.VMEM((D2, H), jnp.float32),
        pltpu.VMEM((NTP, H), jnp.float32),
        pltpu.VMEM_SHARED((NTP, H), jnp.float32),
        pltpu.VMEM((2 * GROUP * U,), jnp.int32),
        pltpu.VMEM((2 * GROUP * U,), jnp.int32),
        pltpu.VMEM((2 * GROUP * U,), jnp.int32),
        pltpu.VMEM((GROUP, U), jnp.int32),
        pltpu.VMEM((GROUP // 2, 2 * U, H), jnp.float32),
        pltpu.SemaphoreType.DMA((GROUP,)),
        pltpu.SemaphoreType.DMA((GROUP // 2,)),
        pltpu.SemaphoreType.DMA((2,)),
    ],
)(_sc_body)


@jax.jit
def kernel(edge_attr, W0, W1, W2):
    ea_t = edge_attr.T
    return _launch(ea_t[0], ea_t[1], ea_t[2], W0, W1, W2)
